# baseline (speedup 1.0000x reference)
import functools

import jax
import jax.numpy as jnp
from jax import lax
from jax.experimental import pallas as pl
from jax.experimental.pallas import tpu as pltpu
from jax.experimental.pallas import tpu_sc as plsc

NUM_SEG = 10000
_BLK = 8192
_NQ = 4

_NC = 2
_NS = 16


def _one_row_block(x, ct, w1_ref, b1_ref, w2_ref, b2_ref):
    h = lax.dot_general(w1_ref[...], x, (((1,), (1,)), ((), ())),
                        preferred_element_type=jnp.float32)
    h = h + b1_ref[...]
    h = 0.5 * h * (jnp.tanh(0.5 * h) + 1.0)
    s = lax.dot_general(w2_ref[...], h, (((1,), (0,)), ((), ())),
                        preferred_element_type=jnp.float32)
    sp = (ct[0:1, :] * ct[0:1, :] + ct[1:2, :] * ct[1:2, :]
          + ct[2:3, :] * ct[2:3, :])
    return ((s + b2_ref[...]) * sp).reshape(_BLK)


def _mlp_body(*refs):
    x_refs = refs[:_NQ]
    ct_refs = refs[_NQ:2 * _NQ]
    w1_ref, b1_ref, w2_ref, b2_ref, o_ref = refs[2 * _NQ:]
    for q in range(_NQ):
        o_ref[pl.ds(q * _BLK, _BLK)] = _one_row_block(
            x_refs[q][...], ct_refs[q][...], w1_ref, b1_ref, w2_ref, b2_ref)


def _combine_body(p_ref, o_ref):
    o_ref[...] = p_ref[0:1, :] + p_ref[1:2, :]


def _sc_seg_sum_body(T, idx_ofs, v_hbm, idx_hbm, zero_hbm, out_hbm,
                     idx_v, val_v, acc_sh):
    c = lax.axis_index("c")
    s = lax.axis_index("s")
    wid = s * _NC + c
    base = wid * T
    pltpu.sync_copy(idx_hbm.at[pl.ds(idx_ofs + base, T)], idx_v)
    pltpu.sync_copy(v_hbm.at[pl.ds(base, T)], val_v)

    @pl.when(s == 0)
    def _():
        pltpu.sync_copy(zero_hbm, acc_sh)

    plsc.subcore_barrier()
    pltpu.sync_copy(val_v, acc_sh.at[idx_v], add=True)
    plsc.subcore_barrier()

    @pl.when(s == 0)
    def _():
        pltpu.sync_copy(acc_sh, out_hbm.at[c])


def _make_sc_seg_sum(rows, idx_ofs):
    T = rows // (_NC * _NS)
    mesh = plsc.VectorSubcoreMesh(core_axis_name="c", subcore_axis_name="s")
    return functools.partial(
        pl.kernel,
        out_type=jax.ShapeDtypeStruct((_NC, NUM_SEG), jnp.float32),
        mesh=mesh,
        scratch_types=[
            pltpu.VMEM((T,), jnp.int32),
            pltpu.VMEM((T,), jnp.float32),
            pltpu.VMEM_SHARED((NUM_SEG,), jnp.float32),
        ],
    )(functools.partial(_sc_seg_sum_body, T, idx_ofs))


def kernel(x_scalar, x_spherical, coord, batch_index, W1, b1, W2, b2):
    n, d = x_scalar.shape
    hdim = W1.shape[0]
    idx = batch_index.astype(jnp.int32)
    ct = coord.T
    b1c = b1.reshape(hdim, 1)
    b2c = b2.reshape(1, 1)

    step = _NQ * _BLK
    nstep = (n + step - 1) // step
    x_specs = [pl.BlockSpec((_BLK, d), functools.partial(
        lambda q, i: (_NQ * i + q, 0), q)) for q in range(_NQ)]
    ct_specs = [pl.BlockSpec((3, _BLK), functools.partial(
        lambda q, i: (0, _NQ * i + q), q)) for q in range(_NQ)]
    v = pl.pallas_call(
        _mlp_body,
        grid=(nstep,),
        in_specs=x_specs + ct_specs + [
            pl.BlockSpec((hdim, d), lambda i: (0, 0)),
            pl.BlockSpec((hdim, 1), lambda i: (0, 0)),
            pl.BlockSpec((1, hdim), lambda i: (0, 0)),
            pl.BlockSpec((1, 1), lambda i: (0, 0)),
        ],
        out_specs=pl.BlockSpec((step,), lambda i: (i,)),
        out_shape=jax.ShapeDtypeStruct((n,), jnp.float32),
    )(*([x_scalar] * _NQ), *([ct] * _NQ), W1, b1c, W2, b2c)

    zeros = jnp.zeros((NUM_SEG,), jnp.float32)
    p = _make_sc_seg_sum(n, 0)(v, idx, zeros)

    out = pl.pallas_call(
        _combine_body,
        out_shape=jax.ShapeDtypeStruct((1, NUM_SEG), jnp.float32),
    )(p)
    return out.reshape(NUM_SEG, 1)


# speedup vs baseline: 1.0028x; 1.0028x over previous
import functools

import jax
import jax.numpy as jnp
from jax import lax
from jax.experimental import pallas as pl
from jax.experimental.pallas import tpu as pltpu
from jax.experimental.pallas import tpu_sc as plsc

NUM_SEG = 10000
_BLK = 16384

_NC = 2
_NS = 16


def _one_row_block(x, ct, w1_ref, b1_ref, w2_ref, b2_ref):
    h = lax.dot_general(w1_ref[...], x, (((1,), (1,)), ((), ())),
                        preferred_element_type=jnp.float32)
    h = h + b1_ref[...]
    h = 0.5 * h * (jnp.tanh(0.5 * h) + 1.0)
    s = lax.dot_general(w2_ref[...], h, (((1,), (0,)), ((), ())),
                        preferred_element_type=jnp.float32)
    sp = (ct[0:1, :] * ct[0:1, :] + ct[1:2, :] * ct[1:2, :]
          + ct[2:3, :] * ct[2:3, :])
    return ((s + b2_ref[...]) * sp).reshape(_BLK)


def _mlp_body(x1_ref, x2_ref, ct1_ref, ct2_ref,
              w1_ref, b1_ref, w2_ref, b2_ref, o_ref):
    o_ref[pl.ds(0, _BLK)] = _one_row_block(
        x1_ref[...], ct1_ref[...], w1_ref, b1_ref, w2_ref, b2_ref)
    o_ref[pl.ds(_BLK, _BLK)] = _one_row_block(
        x2_ref[...], ct2_ref[...], w1_ref, b1_ref, w2_ref, b2_ref)


def _combine_body(p_ref, o_ref):
    o_ref[...] = p_ref[0:1, :] + p_ref[1:2, :]


def _sc_seg_sum_body(T, idx_ofs, v_hbm, idx_hbm, zero_hbm, out_hbm,
                     idx_v, val_v, acc_sh):
    c = lax.axis_index("c")
    s = lax.axis_index("s")
    wid = s * _NC + c
    base = wid * T
    pltpu.sync_copy(idx_hbm.at[pl.ds(idx_ofs + base, T)], idx_v)
    pltpu.sync_copy(v_hbm.at[pl.ds(base, T)], val_v)

    @pl.when(s == 0)
    def _():
        pltpu.sync_copy(zero_hbm, acc_sh)

    plsc.subcore_barrier()
    pltpu.sync_copy(val_v, acc_sh.at[idx_v], add=True)
    plsc.subcore_barrier()

    @pl.when(s == 0)
    def _():
        pltpu.sync_copy(acc_sh, out_hbm.at[c])


def _make_sc_seg_sum(rows, idx_ofs):
    T = rows // (_NC * _NS)
    mesh = plsc.VectorSubcoreMesh(core_axis_name="c", subcore_axis_name="s")
    return functools.partial(
        pl.kernel,
        out_type=jax.ShapeDtypeStruct((_NC, NUM_SEG), jnp.float32),
        mesh=mesh,
        scratch_types=[
            pltpu.VMEM((T,), jnp.int32),
            pltpu.VMEM((T,), jnp.float32),
            pltpu.VMEM_SHARED((NUM_SEG,), jnp.float32),
        ],
    )(functools.partial(_sc_seg_sum_body, T, idx_ofs))


def kernel(x_scalar, x_spherical, coord, batch_index, W1, b1, W2, b2):
    n, d = x_scalar.shape
    hdim = W1.shape[0]
    idx = batch_index.astype(jnp.int32)
    ct = coord.T
    b1c = b1.reshape(hdim, 1)
    b2c = b2.reshape(1, 1)

    step = 2 * _BLK
    nstep = (n + step - 1) // step
    v = pl.pallas_call(
        _mlp_body,
        grid=(nstep,),
        in_specs=[
            pl.BlockSpec((_BLK, d), lambda i: (2 * i, 0)),
            pl.BlockSpec((_BLK, d), lambda i: (2 * i + 1, 0)),
            pl.BlockSpec((3, _BLK), lambda i: (0, 2 * i)),
            pl.BlockSpec((3, _BLK), lambda i: (0, 2 * i + 1)),
            pl.BlockSpec((hdim, d), lambda i: (0, 0)),
            pl.BlockSpec((hdim, 1), lambda i: (0, 0)),
            pl.BlockSpec((1, hdim), lambda i: (0, 0)),
            pl.BlockSpec((1, 1), lambda i: (0, 0)),
        ],
        out_specs=pl.BlockSpec((step,), lambda i: (i,)),
        out_shape=jax.ShapeDtypeStruct((n,), jnp.float32),
    )(x_scalar, x_scalar, ct, ct, W1, b1c, W2, b2c)

    zeros = jnp.zeros((NUM_SEG,), jnp.float32)
    p = _make_sc_seg_sum(n, 0)(v, idx, zeros)

    out = pl.pallas_call(
        _combine_body,
        out_shape=jax.ShapeDtypeStruct((1, NUM_SEG), jnp.float32),
    )(p)
    return out.reshape(NUM_SEG, 1)
